# baseline (speedup 1.0000x reference)
import functools

import jax
import jax.numpy as jnp
from jax import lax
from jax.experimental import pallas as pl
from jax.experimental.pallas import tpu as pltpu

N = 50000
B = 512


TAIL_BM = 128


def _tail_body(g_ref, tt_ref, wg1_ref, bg1_ref, wg2_ref, bg2_ref,
               wc2_ref, bc_ref, wxt3_ref, bxt_ref,
               wf1_ref, bf1_ref, wf2_ref, bf2_ref, wo_ref, bo_ref,
               out_ref):
    f32 = jnp.float32
    bm = TAIL_BM
    g = g_ref[...]
    g1 = jax.nn.relu(jnp.dot(g, wg1_ref[...], preferred_element_type=f32)
                     + bg1_ref[...])
    g2 = jnp.dot(g1, wg2_ref[...], preferred_element_type=f32) + bg2_ref[...]

    xt = bxt_ref[...] * jnp.ones((bm, 1), f32)
    for t in range(12):
        ct = bc_ref[...] * jnp.ones((bm, 1), f32)
        for k in range(8):
            ct = ct + jnp.dot(tt_ref[:, t + k, :], wc2_ref[k],
                              preferred_element_type=f32)
        xt = xt + jnp.dot(ct, wxt3_ref[t], preferred_element_type=f32)

    h = jnp.dot(g2, wf1_ref[0:128, :], preferred_element_type=f32)
    h = h + jnp.dot(xt, wf1_ref[128:256, :], preferred_element_type=f32)
    h = jax.nn.relu(h + bf1_ref[...])
    h = jax.nn.relu(jnp.dot(h, wf2_ref[...], preferred_element_type=f32)
                    + bf2_ref[...])
    out_ref[...] = jnp.dot(h, wo_ref[...], preferred_element_type=f32) + bo_ref[...]


def _tail(g, target, Wg1, bg1, Wg2, bg2, Wc, bc, Wxt, bxt,
          Wf1, bf1, Wf2, bf2, Wo, bo):
    tt = jnp.transpose(target, (0, 2, 1))
    wc2 = jnp.transpose(Wc, (2, 1, 0))
    wxt3 = jnp.transpose(Wxt.reshape(32, 12, 128), (1, 0, 2))
    bm = TAIL_BM
    row_spec = lambda minor: pl.BlockSpec((bm,) + minor, lambda i: (i,) + (0,) * len(minor))
    full = lambda a: pl.BlockSpec(a.shape, lambda i: (0,) * a.ndim)
    weights = [Wg1, bg1.reshape(1, -1), Wg2, bg2.reshape(1, -1),
               wc2, bc.reshape(1, -1), wxt3, bxt.reshape(1, -1),
               Wf1, bf1.reshape(1, -1), Wf2, bf2.reshape(1, -1),
               Wo, bo.reshape(1, -1)]
    return pl.pallas_call(
        _tail_body,
        grid=(B // bm,),
        in_specs=[row_spec((g.shape[1],)), row_spec((19, 750))]
                 + [full(w) for w in weights],
        out_specs=row_spec((1,)),
        out_shape=jax.ShapeDtypeStruct((B, 1), jnp.float32),
    )(g, tt, *weights)


def _conv(x, src, dst, W, b):
    n = x.shape[0]
    h = x @ W
    deg = jnp.zeros((n,), x.dtype).at[dst].add(1.0)
    dinv = jnp.where(deg > 0, jax.lax.rsqrt(jnp.maximum(deg, 1e-12)), 0.0)
    norm = dinv[src] * dinv[dst]
    out = jnp.zeros((n, W.shape[1]), x.dtype).at[dst].add(h[src] * norm[:, None])
    return out + b


def kernel(x, edge_index, batch, target, W1, b1, W2, b2, W3, b3, Wg1, bg1,
           Wg2, bg2, Wc, bc, Wxt, bxt, Wf1, bf1, Wf2, bf2, Wo, bo):
    n = x.shape[0]
    loop = jnp.arange(n, dtype=edge_index.dtype)
    src = jnp.concatenate([edge_index[0], loop])
    dst = jnp.concatenate([edge_index[1], loop])
    h = jax.nn.relu(_conv(x, src, dst, W1, b1))
    h = jax.nn.relu(_conv(h, src, dst, W2, b2))
    h = jax.nn.relu(_conv(h, src, dst, W3, b3))
    g = jax.ops.segment_max(h, batch, num_segments=B)
    g = jnp.where(jnp.isfinite(g), g, 0.0)
    return _tail(g, target, Wg1, bg1, Wg2, bg2, Wc, bc, Wxt, bxt,
                 Wf1, bf1, Wf2, bf2, Wo, bo)


# speedup vs baseline: 4.8851x; 4.8851x over previous
import functools

import jax
import jax.numpy as jnp
from jax import lax
from jax.experimental import pallas as pl
from jax.experimental.pallas import tpu as pltpu
from jax.experimental.pallas import tpu_sc as plsc

N = 50000
B = 512
E = 800000
EPT = 25088
EPAD = 32 * EPT
PADV = 1 << 30
K = 128
NROW = 50176


DUMP = NROW
ACC_ROWS = NROW + 32
ZCHD = ACC_ROWS // 16
WCHD = NROW // 16
NBATCH = EPT // K

_MESH = plsc.VectorSubcoreMesh(core_axis_name="c", subcore_axis_name="s")
_SC_PARAMS = pltpu.CompilerParams(use_tc_tiling_on_sc=False)


SB = 3584
NBLK = EPT // SB
BPB = SB // K


def _deg_body(dst_hbm, zeros_hbm, ones_hbm, out_hbm, dbuf, fdst, ones, acc):
    c = lax.axis_index("c")
    s = lax.axis_index("s")
    wid = s * 2 + c
    pltpu.sync_copy(ones_hbm, ones)
    pltpu.sync_copy(zeros_hbm, acc.at[pl.ds(s * ZCHD, ZCHD)])
    plsc.subcore_barrier()

    def block(blk, carry):
        pltpu.sync_copy(dst_hbm.at[wid, pl.ds(blk * SB, SB)], dbuf)

        def batch(j, carry2):
            for t in range(8):
                dv = dbuf[pl.ds(j * K + t * 16, 16)]
                fdst[pl.ds(t * 16, 16)] = jnp.minimum(dv, DUMP)
            pltpu.sync_copy(ones, acc.at[fdst], add=True)
            return carry2

        lax.fori_loop(0, BPB, batch, 0)
        return carry

    lax.fori_loop(0, NBLK, block, 0)
    plsc.subcore_barrier()
    pltpu.sync_copy(acc.at[pl.ds(s * WCHD, WCHD)],
                    out_hbm.at[c, pl.ds(s * WCHD, WCHD)])


def _sc_degree(dstp):
    f32 = jnp.float32
    fn = pl.kernel(
        _deg_body,
        out_type=jax.ShapeDtypeStruct((2, NROW, 16), f32),
        mesh=_MESH,
        compiler_params=_SC_PARAMS,
        scratch_types=[
            pltpu.VMEM((SB,), jnp.int32),
            pltpu.VMEM((K,), jnp.int32),
            pltpu.VMEM((K, 16), f32),
            pltpu.VMEM_SHARED((ACC_ROWS, 16), f32),
        ],
    )
    zeros = jnp.zeros((ZCHD, 16), f32)
    ones = jnp.ones((K, 16), f32)
    return fn(dstp, zeros, ones)


def _agg_body(ncc, src_hbm, dst_hbm, hp_hbm, zeros_hbm, out_hbm,
              sbuf, dbuf, fsrc, fdst, rowbuf, acc):
    c = lax.axis_index("c")
    s = lax.axis_index("s")
    wid = s * 2 + c
    for cc in range(ncc):
        pltpu.sync_copy(zeros_hbm, acc.at[pl.ds(s * ZCHD, ZCHD)])
        plsc.subcore_barrier()

        def block(blk, carry):
            pltpu.sync_copy(src_hbm.at[wid, pl.ds(blk * SB, SB)], sbuf)
            pltpu.sync_copy(dst_hbm.at[wid, pl.ds(blk * SB, SB)], dbuf)

            def batch(j, carry2):
                for t in range(8):
                    sv = sbuf[pl.ds(j * K + t * 16, 16)]
                    dv = dbuf[pl.ds(j * K + t * 16, 16)]
                    fsrc[pl.ds(t * 16, 16)] = sv + cc * NROW
                    fdst[pl.ds(t * 16, 16)] = jnp.minimum(dv, DUMP)
                pltpu.sync_copy(hp_hbm.at[fsrc], rowbuf)
                pltpu.sync_copy(rowbuf, acc.at[fdst], add=True)
                return carry2

            lax.fori_loop(0, BPB, batch, 0)
            return carry

        lax.fori_loop(0, NBLK, block, 0)
        plsc.subcore_barrier()
        pltpu.sync_copy(acc.at[pl.ds(s * WCHD, WCHD)],
                        out_hbm.at[c, cc, pl.ds(s * WCHD, WCHD)])
        plsc.subcore_barrier()


def _make_sc_agg(ncc):
    f32 = jnp.float32
    fn = pl.kernel(
        functools.partial(_agg_body, ncc),
        out_type=jax.ShapeDtypeStruct((2, ncc, NROW, 32), f32),
        mesh=_MESH,
        compiler_params=_SC_PARAMS,
        scratch_types=[
            pltpu.VMEM((SB,), jnp.int32),
            pltpu.VMEM((SB,), jnp.int32),
            pltpu.VMEM((K,), jnp.int32),
            pltpu.VMEM((K,), jnp.int32),
            pltpu.VMEM((K, 32), f32),
            pltpu.VMEM_SHARED((ACC_ROWS, 32), f32),
        ],
    )

    def run(srcp, dstp, hp2):
        zeros = jnp.zeros((ZCHD, 32), f32)
        return fn(srcp, dstp, hp2, zeros)

    return run


_SC_AGG = {ncc: _make_sc_agg(ncc) for ncc in (3, 5, 10)}


TAIL_BM = 128


def _tail_body(g_ref, tt_ref, wg1_ref, bg1_ref, wg2_ref, bg2_ref,
               wc2_ref, bc_ref, wxt3_ref, bxt_ref,
               wf1_ref, bf1_ref, wf2_ref, bf2_ref, wo_ref, bo_ref,
               out_ref):
    f32 = jnp.float32
    bm = TAIL_BM
    g = g_ref[...]
    g1 = jax.nn.relu(jnp.dot(g, wg1_ref[...], preferred_element_type=f32)
                     + bg1_ref[...])
    g2 = jnp.dot(g1, wg2_ref[...], preferred_element_type=f32) + bg2_ref[...]

    xt = bxt_ref[...] * jnp.ones((bm, 1), f32)
    for t in range(12):
        ct = bc_ref[...] * jnp.ones((bm, 1), f32)
        for k in range(8):
            ct = ct + jnp.dot(tt_ref[:, t + k, :], wc2_ref[k],
                              preferred_element_type=f32)
        xt = xt + jnp.dot(ct, wxt3_ref[t], preferred_element_type=f32)

    h = jnp.dot(g2, wf1_ref[0:128, :], preferred_element_type=f32)
    h = h + jnp.dot(xt, wf1_ref[128:256, :], preferred_element_type=f32)
    h = jax.nn.relu(h + bf1_ref[...])
    h = jax.nn.relu(jnp.dot(h, wf2_ref[...], preferred_element_type=f32)
                    + bf2_ref[...])
    out_ref[...] = jnp.dot(h, wo_ref[...], preferred_element_type=f32) + bo_ref[...]


def _tail(g, target, Wg1, bg1, Wg2, bg2, Wc, bc, Wxt, bxt,
          Wf1, bf1, Wf2, bf2, Wo, bo):
    tt = jnp.transpose(target, (0, 2, 1))
    wc2 = jnp.transpose(Wc, (2, 1, 0))
    wxt3 = jnp.transpose(Wxt.reshape(32, 12, 128), (1, 0, 2))
    bm = TAIL_BM
    row_spec = lambda minor: pl.BlockSpec((bm,) + minor, lambda i: (i,) + (0,) * len(minor))
    full = lambda a: pl.BlockSpec(a.shape, lambda i: (0,) * a.ndim)
    weights = [Wg1, bg1.reshape(1, -1), Wg2, bg2.reshape(1, -1),
               wc2, bc.reshape(1, -1), wxt3, bxt.reshape(1, -1),
               Wf1, bf1.reshape(1, -1), Wf2, bf2.reshape(1, -1),
               Wo, bo.reshape(1, -1)]
    return pl.pallas_call(
        _tail_body,
        grid=(B // bm,),
        in_specs=[row_spec((g.shape[1],)), row_spec((19, 750))]
                 + [full(w) for w in weights],
        out_specs=row_spec((1,)),
        out_shape=jax.ShapeDtypeStruct((B, 1), jnp.float32),
    )(g, tt, *weights)


def _conv_agg(h, srcp, dstp, dinv):
    n, f = h.shape
    ncc = (f + 31) // 32
    fp = ncc * 32
    hp = h * dinv[:, None]
    hpad = jnp.pad(hp, ((0, NROW - n), (0, fp - f)))
    hp2 = hpad.reshape(NROW, ncc, 32).transpose(1, 0, 2).reshape(ncc * NROW, 32)
    aggp = _SC_AGG[ncc](srcp, dstp, hp2)
    agg = (aggp[0] + aggp[1]).transpose(1, 0, 2).reshape(NROW, fp)[:n, :f]
    return dinv[:, None] * (agg + hp)


def kernel(x, edge_index, batch, target, W1, b1, W2, b2, W3, b3, Wg1, bg1,
           Wg2, bg2, Wc, bc, Wxt, bxt, Wf1, bf1, Wf2, bf2, Wo, bo):
    n = x.shape[0]
    src = edge_index[0]
    dst = edge_index[1]
    srcp = jnp.concatenate(
        [src, jnp.zeros((EPAD - E,), jnp.int32)]).reshape(32, EPT)
    dstp = jnp.concatenate(
        [dst, jnp.full((EPAD - E,), PADV, jnp.int32)]).reshape(32, EPT)

    degp = _sc_degree(dstp)
    deg = degp[0, :n, 0] + degp[1, :n, 0] + 1.0
    dinv = lax.rsqrt(deg)

    h = jax.nn.relu(_conv_agg(x @ W1, srcp, dstp, dinv) + b1)
    h = jax.nn.relu(_conv_agg(h @ W2, srcp, dstp, dinv) + b2)
    h = jax.nn.relu(_conv_agg(h @ W3, srcp, dstp, dinv) + b3)
    g = jax.ops.segment_max(h, batch, num_segments=B)
    g = jnp.where(jnp.isfinite(g), g, 0.0)
    return _tail(g, target, Wg1, bg1, Wg2, bg2, Wc, bc, Wxt, bxt,
                 Wf1, bf1, Wf2, bf2, Wo, bo)


# speedup vs baseline: 5.0948x; 1.0429x over previous
import functools

import jax
import jax.numpy as jnp
from jax import lax
from jax.experimental import pallas as pl
from jax.experimental.pallas import tpu as pltpu
from jax.experimental.pallas import tpu_sc as plsc

N = 50000
B = 512
E = 800000
EPT = 25088
EPAD = 32 * EPT
PADV = 1 << 30
K = 128
NROW = 50176


DUMP = NROW
ACC_ROWS = NROW + 32
ZCHD = ACC_ROWS // 16
WCHD = NROW // 16
NBATCH = EPT // K

_MESH = plsc.VectorSubcoreMesh(core_axis_name="c", subcore_axis_name="s")
_SC_PARAMS = pltpu.CompilerParams(use_tc_tiling_on_sc=False)


SB = 3584
NBLK = EPT // SB
BPB = SB // K


def _deg_body(dst_hbm, zeros_hbm, ones_hbm, out_hbm, dbuf, fdst, ones, acc):
    c = lax.axis_index("c")
    s = lax.axis_index("s")
    wid = s * 2 + c
    pltpu.sync_copy(ones_hbm, ones)
    pltpu.sync_copy(zeros_hbm, acc.at[pl.ds(s * ZCHD, ZCHD)])
    plsc.subcore_barrier()

    def block(blk, carry):
        pltpu.sync_copy(dst_hbm.at[wid, pl.ds(blk * SB, SB)], dbuf)

        def batch(j, carry2):
            for t in range(8):
                dv = dbuf[pl.ds(j * K + t * 16, 16)]
                fdst[pl.ds(t * 16, 16)] = jnp.minimum(dv, DUMP)
            pltpu.sync_copy(ones, acc.at[fdst], add=True)
            return carry2

        lax.fori_loop(0, BPB, batch, 0)
        return carry

    lax.fori_loop(0, NBLK, block, 0)
    plsc.subcore_barrier()
    pltpu.sync_copy(acc.at[pl.ds(s * WCHD, WCHD)],
                    out_hbm.at[c, pl.ds(s * WCHD, WCHD)])


def _sc_degree(dstp):
    f32 = jnp.float32
    fn = pl.kernel(
        _deg_body,
        out_type=jax.ShapeDtypeStruct((2, NROW, 16), f32),
        mesh=_MESH,
        compiler_params=_SC_PARAMS,
        scratch_types=[
            pltpu.VMEM((SB,), jnp.int32),
            pltpu.VMEM((K,), jnp.int32),
            pltpu.VMEM((K, 16), f32),
            pltpu.VMEM_SHARED((ACC_ROWS, 16), f32),
        ],
    )
    zeros = jnp.zeros((ZCHD, 16), f32)
    ones = jnp.ones((K, 16), f32)
    return fn(dstp, zeros, ones)


def _agg_body(ncc, src_hbm, dst_hbm, hp_hbm, zeros_hbm, out_hbm,
              sbuf, dbuf, fsrc, fdst, rowbuf, acc):
    c = lax.axis_index("c")
    s = lax.axis_index("s")
    wid = s * 2 + c
    for cc in range(ncc):
        pltpu.sync_copy(zeros_hbm, acc.at[pl.ds(s * ZCHD, ZCHD)])
        plsc.subcore_barrier()

        def block(blk, carry):
            pltpu.sync_copy(src_hbm.at[wid, pl.ds(blk * SB, SB)], sbuf)
            pltpu.sync_copy(dst_hbm.at[wid, pl.ds(blk * SB, SB)], dbuf)

            def batch(j, carry2):
                for t in range(8):
                    sv = sbuf[pl.ds(j * K + t * 16, 16)]
                    dv = dbuf[pl.ds(j * K + t * 16, 16)]
                    fsrc[pl.ds(t * 16, 16)] = sv + cc * NROW
                    fdst[pl.ds(t * 16, 16)] = jnp.minimum(dv, DUMP)
                pltpu.sync_copy(hp_hbm.at[fsrc], rowbuf)
                pltpu.sync_copy(rowbuf, acc.at[fdst], add=True)
                return carry2

            lax.fori_loop(0, BPB, batch, 0)
            return carry

        lax.fori_loop(0, NBLK, block, 0)
        plsc.subcore_barrier()
        pltpu.sync_copy(acc.at[pl.ds(s * WCHD, WCHD)],
                        out_hbm.at[c, cc, pl.ds(s * WCHD, WCHD)])
        plsc.subcore_barrier()


def _make_sc_agg(ncc):
    f32 = jnp.float32
    fn = pl.kernel(
        functools.partial(_agg_body, ncc),
        out_type=jax.ShapeDtypeStruct((2, ncc, NROW, 32), f32),
        mesh=_MESH,
        compiler_params=_SC_PARAMS,
        scratch_types=[
            pltpu.VMEM((SB,), jnp.int32),
            pltpu.VMEM((SB,), jnp.int32),
            pltpu.VMEM((K,), jnp.int32),
            pltpu.VMEM((K,), jnp.int32),
            pltpu.VMEM((K, 32), f32),
            pltpu.VMEM_SHARED((ACC_ROWS, 32), f32),
        ],
    )

    def run(srcp, dstp, hp2):
        zeros = jnp.zeros((ZCHD, 32), f32)
        return fn(srcp, dstp, hp2, zeros)

    return run


_SC_AGG = {ncc: _make_sc_agg(ncc) for ncc in (3, 5, 10)}


TAIL_BM = 64


def _tail_body(g_ref, tt_ref, wg1_ref, bg1_ref, wg2_ref, bg2_ref,
               wc2_ref, bc_ref, wxt3_ref, bxt_ref,
               wf1_ref, bf1_ref, wf2_ref, bf2_ref, wo_ref, bo_ref,
               out_ref):
    f32 = jnp.float32
    bm = TAIL_BM
    g = g_ref[...]
    g = jnp.where(jnp.isfinite(g), g, 0.0)
    g1 = jax.nn.relu(jnp.dot(g, wg1_ref[...], preferred_element_type=f32)
                     + bg1_ref[...])
    g2 = jnp.dot(g1, wg2_ref[...], preferred_element_type=f32) + bg2_ref[...]

    xt = bxt_ref[...] * jnp.ones((bm, 1), f32)
    for t in range(12):
        ct = bc_ref[...] * jnp.ones((bm, 1), f32)
        for k in range(8):
            ct = ct + jnp.dot(tt_ref[:, t + k, :], wc2_ref[k],
                              preferred_element_type=f32)
        xt = xt + jnp.dot(ct, wxt3_ref[t], preferred_element_type=f32)

    h = jnp.dot(g2, wf1_ref[0:128, :], preferred_element_type=f32)
    h = h + jnp.dot(xt, wf1_ref[128:256, :], preferred_element_type=f32)
    h = jax.nn.relu(h + bf1_ref[...])
    h = jax.nn.relu(jnp.dot(h, wf2_ref[...], preferred_element_type=f32)
                    + bf2_ref[...])
    out_ref[...] = jnp.dot(h, wo_ref[...], preferred_element_type=f32) + bo_ref[...]


def _tail(g, target, Wg1, bg1, Wg2, bg2, Wc, bc, Wxt, bxt,
          Wf1, bf1, Wf2, bf2, Wo, bo):
    tt = jnp.transpose(target, (0, 2, 1))
    wc2 = jnp.transpose(Wc, (2, 1, 0))
    wxt3 = jnp.transpose(Wxt.reshape(32, 12, 128), (1, 0, 2))
    wg1p = jnp.pad(Wg1, ((0, 320 - Wg1.shape[0]), (0, 0)))
    bm = TAIL_BM
    row_spec = lambda minor: pl.BlockSpec((bm,) + minor, lambda i: (i,) + (0,) * len(minor))
    full = lambda a: pl.BlockSpec(a.shape, lambda i: (0,) * a.ndim)
    weights = [wg1p, bg1.reshape(1, -1), Wg2, bg2.reshape(1, -1),
               wc2, bc.reshape(1, -1), wxt3, bxt.reshape(1, -1),
               Wf1, bf1.reshape(1, -1), Wf2, bf2.reshape(1, -1),
               Wo, bo.reshape(1, -1)]
    return pl.pallas_call(
        _tail_body,
        grid=(B // bm,),
        in_specs=[row_spec((g.shape[1],)), row_spec((19, 750))]
                 + [full(w) for w in weights],
        out_specs=row_spec((1,)),
        out_shape=jax.ShapeDtypeStruct((B, 1), jnp.float32),
    )(g, tt, *weights)


TCBLK = 896
TCGRID = NROW // TCBLK


def _rows(minor):
    return pl.BlockSpec((TCBLK,) + minor, lambda i: (i,) + (0,) * len(minor))


def _full(a):
    return pl.BlockSpec(a.shape, lambda i: (0,) * a.ndim)


def _chunks(ncc):
    return pl.BlockSpec((ncc, TCBLK, 32), lambda i: (0, i, 0))


def _chunks2(ncc):
    return pl.BlockSpec((2, ncc, TCBLK, 32), lambda i: (0, 0, i, 0))


def _cat(ag):
    return jnp.concatenate([ag[c] for c in range(ag.shape[0])], axis=1)


def _tc1_body(x_ref, deg_ref, w_ref, hp_ref, dinv_ref):
    deg = deg_ref[0, :, 0:1] + deg_ref[1, :, 0:1] + 1.0
    dinv = lax.rsqrt(deg)
    h = jnp.dot(x_ref[...], w_ref[...], preferred_element_type=jnp.float32)
    h = h * dinv
    for c in range(hp_ref.shape[0]):
        hp_ref[c] = h[:, c * 32:(c + 1) * 32]
    dinv_ref[...] = dinv


def _tc_mid_body(ncc_out, a_ref, hp_ref, dinv_ref, w_ref, b_ref, out_ref):
    ag = _cat(a_ref[0] + a_ref[1]) + _cat(hp_ref[...])
    dinv = dinv_ref[...]
    h = jax.nn.relu(dinv * ag + b_ref[...])
    h2 = jnp.dot(h, w_ref[...], preferred_element_type=jnp.float32) * dinv
    for c in range(ncc_out):
        out_ref[c] = h2[:, c * 32:(c + 1) * 32]


def _tc4_body(a_ref, hp_ref, dinv_ref, b_ref, out_ref):
    ag = _cat(a_ref[0] + a_ref[1]) + _cat(hp_ref[...])
    out_ref[...] = jax.nn.relu(dinv_ref[...] * ag + b_ref[...])


def _tcseg_body(h_ref, ids_ref, out_ref, acc_ref, prev_ref):
    step = pl.program_id(0)

    @pl.when(step == 0)
    def _():
        out_ref[...] = jnp.full(out_ref.shape, -jnp.inf, jnp.float32)
        acc_ref[...] = jnp.full(acc_ref.shape, -jnp.inf, jnp.float32)
        prev_ref[0] = ids_ref[0, 0]

    def rowbody(r, carry):
        rid = ids_ref[step, r]
        prev = prev_ref[0]

        @pl.when(rid != prev)
        def _():
            out_ref[pl.ds(prev, 1), :] = acc_ref[...]
            acc_ref[...] = jnp.full(acc_ref.shape, -jnp.inf, jnp.float32)

        acc_ref[...] = jnp.maximum(acc_ref[...], h_ref[pl.ds(r, 1), :])
        prev_ref[0] = rid
        return carry

    lax.fori_loop(0, TCBLK, rowbody, 0)

    @pl.when(step == TCGRID - 1)
    def _():
        out_ref[pl.ds(prev_ref[0], 1), :] = acc_ref[...]


def kernel(x, edge_index, batch, target, W1, b1, W2, b2, W3, b3, Wg1, bg1,
           Wg2, bg2, Wc, bc, Wxt, bxt, Wf1, bf1, Wf2, bf2, Wo, bo):
    f32 = jnp.float32
    n = x.shape[0]
    srcp = jnp.concatenate(
        [edge_index[0], jnp.zeros((EPAD - E,), jnp.int32)]).reshape(32, EPT)
    dstp = jnp.concatenate(
        [edge_index[1], jnp.full((EPAD - E,), PADV, jnp.int32)]).reshape(32, EPT)
    xp = jnp.pad(x, ((0, NROW - n), (0, 0)))
    batchp = jnp.concatenate([batch, jnp.full((NROW - n,), B, jnp.int32)])

    w1p = jnp.pad(W1, ((0, 0), (0, 96 - 78)))
    w2p = jnp.pad(W2, ((0, 96 - 78), (0, 160 - 156)))
    w3p = jnp.pad(W3, ((0, 160 - 156), (0, 320 - 312)))
    b1p = jnp.pad(b1, (0, 96 - 78)).reshape(1, -1)
    b2p = jnp.pad(b2, (0, 160 - 156)).reshape(1, -1)
    b3p = jnp.pad(b3, (0, 320 - 312)).reshape(1, -1)

    degp = _sc_degree(dstp)

    hp1, dinv = pl.pallas_call(
        _tc1_body,
        grid=(TCGRID,),
        in_specs=[_rows((78,)), pl.BlockSpec((2, TCBLK, 16), lambda i: (0, i, 0)),
                  _full(w1p)],
        out_specs=[_chunks(3), _rows((1,))],
        out_shape=[jax.ShapeDtypeStruct((3, NROW, 32), f32),
                   jax.ShapeDtypeStruct((NROW, 1), f32)],
    )(xp, degp, w1p)

    agg1 = _SC_AGG[3](srcp, dstp, hp1.reshape(3 * NROW, 32))

    hp2 = pl.pallas_call(
        functools.partial(_tc_mid_body, 5),
        grid=(TCGRID,),
        in_specs=[_chunks2(3), _chunks(3), _rows((1,)), _full(w2p), _full(b1p)],
        out_specs=_chunks(5),
        out_shape=jax.ShapeDtypeStruct((5, NROW, 32), f32),
    )(agg1, hp1, dinv, w2p, b1p)

    agg2 = _SC_AGG[5](srcp, dstp, hp2.reshape(5 * NROW, 32))

    hp3 = pl.pallas_call(
        functools.partial(_tc_mid_body, 10),
        grid=(TCGRID,),
        in_specs=[_chunks2(5), _chunks(5), _rows((1,)), _full(w3p), _full(b2p)],
        out_specs=_chunks(10),
        out_shape=jax.ShapeDtypeStruct((10, NROW, 32), f32),
    )(agg2, hp2, dinv, w3p, b2p)

    agg3 = _SC_AGG[10](srcp, dstp, hp3.reshape(10 * NROW, 32))

    h3f = pl.pallas_call(
        _tc4_body,
        grid=(TCGRID,),
        in_specs=[_chunks2(10), _chunks(10), _rows((1,)), _full(b3p)],
        out_specs=_rows((320,)),
        out_shape=jax.ShapeDtypeStruct((NROW, 320), f32),
    )(agg3, hp3, dinv, b3p)

    g = pl.pallas_call(
        _tcseg_body,
        grid=(TCGRID,),
        in_specs=[_rows((320,)),
                  pl.BlockSpec((TCGRID, TCBLK), lambda i: (0, 0),
                               memory_space=pltpu.SMEM)],
        out_specs=pl.BlockSpec((B + 8, 320), lambda i: (0, 0)),
        out_shape=jax.ShapeDtypeStruct((B + 8, 320), f32),
        scratch_shapes=[pltpu.VMEM((1, 320), f32), pltpu.SMEM((1,), jnp.int32)],
    )(h3f, batchp.reshape(TCGRID, TCBLK))

    return _tail(g, target, Wg1, bg1, Wg2, bg2, Wc, bc, Wxt, bxt,
                 Wf1, bf1, Wf2, bf2, Wo, bo)


# speedup vs baseline: 6.9762x; 1.3693x over previous
import functools

import jax
import jax.numpy as jnp
from jax import lax
from jax.experimental import pallas as pl
from jax.experimental.pallas import tpu as pltpu
from jax.experimental.pallas import tpu_sc as plsc

N = 50000
B = 512
E = 800000
EPT = 25088
EPAD = 32 * EPT
PADV = 1 << 30
K = 128
NROW = 50176


DUMP = NROW
ACC_ROWS = NROW + 32
ZCHD = ACC_ROWS // 16
WCHD = NROW // 16
NBATCH = EPT // K

_MESH = plsc.VectorSubcoreMesh(core_axis_name="c", subcore_axis_name="s")
_SC_PARAMS = pltpu.CompilerParams(use_tc_tiling_on_sc=False)


SB = 3584
NBLK = EPT // SB
BPB = SB // K


def _deg_body(dst_hbm, zeros_hbm, ones_hbm, out_hbm, dbuf, fdst, ones, acc):
    c = lax.axis_index("c")
    s = lax.axis_index("s")
    wid = s * 2 + c
    pltpu.sync_copy(ones_hbm, ones)
    pltpu.sync_copy(zeros_hbm, acc.at[pl.ds(s * ZCHD, ZCHD)])
    plsc.subcore_barrier()

    def block(blk, carry):
        pltpu.sync_copy(dst_hbm.at[wid, pl.ds(blk * SB, SB)], dbuf)

        def batch(j, carry2):
            for t in range(8):
                dv = dbuf[pl.ds(j * K + t * 16, 16)]
                fdst[pl.ds(t * 16, 16)] = jnp.minimum(dv, DUMP)
            pltpu.sync_copy(ones, acc.at[fdst], add=True)
            return carry2

        lax.fori_loop(0, BPB, batch, 0)
        return carry

    lax.fori_loop(0, NBLK, block, 0)
    plsc.subcore_barrier()
    pltpu.sync_copy(acc.at[pl.ds(s * WCHD, WCHD)],
                    out_hbm.at[c, pl.ds(s * WCHD, WCHD)])


def _sc_degree(dstp):
    f32 = jnp.float32
    fn = pl.kernel(
        _deg_body,
        out_type=jax.ShapeDtypeStruct((2, NROW, 16), f32),
        mesh=_MESH,
        compiler_params=_SC_PARAMS,
        scratch_types=[
            pltpu.VMEM((SB,), jnp.int32),
            pltpu.VMEM((K,), jnp.int32),
            pltpu.VMEM((K, 16), f32),
            pltpu.VMEM_SHARED((ACC_ROWS, 16), f32),
        ],
    )
    zeros = jnp.zeros((ZCHD, 16), f32)
    ones = jnp.ones((K, 16), f32)
    return fn(dstp, zeros, ones)


GK = 4


def _agg_body(ncc, src_hbm, dst_hbm, hp_hbm, zeros_hbm, out_hbm,
              sbuf, dbuf, fs0, fs1, fs2, fs3, fd0, fd1, fd2, fd3,
              rb0, rb1, rb2, rb3, gsem, ssem, acc):
    FS = (fs0, fs1, fs2, fs3)
    FD = (fd0, fd1, fd2, fd3)
    RB = (rb0, rb1, rb2, rb3)
    c = lax.axis_index("c")
    s = lax.axis_index("s")
    wid = s * 2 + c
    for cc in range(ncc):
        pltpu.sync_copy(zeros_hbm, acc.at[pl.ds(s * ZCHD, ZCHD)])
        plsc.subcore_barrier()

        def block(blk, carry):
            pltpu.sync_copy(src_hbm.at[wid, pl.ds(blk * SB, SB)], sbuf)
            pltpu.sync_copy(dst_hbm.at[wid, pl.ds(blk * SB, SB)], dbuf)

            def group(gj, carry2):
                base = gj * (GK * K)
                for t in range(GK):
                    for u in range(8):
                        sv = sbuf[pl.ds(base + t * K + u * 16, 16)]
                        dv = dbuf[pl.ds(base + t * K + u * 16, 16)]
                        FS[t][pl.ds(u * 16, 16)] = sv + cc * NROW
                        FD[t][pl.ds(u * 16, 16)] = jnp.minimum(dv, DUMP)
                gds = [pltpu.async_copy(hp_hbm.at[FS[t]], RB[t], gsem)
                       for t in range(GK)]
                sds = []
                for t in range(GK):
                    gds[t].wait()
                    sds.append(pltpu.async_copy(RB[t], acc.at[FD[t]], ssem,
                                                add=True))
                for d in sds:
                    d.wait()
                return carry2

            lax.fori_loop(0, BPB // GK, group, 0)
            return carry

        lax.fori_loop(0, NBLK, block, 0)
        plsc.subcore_barrier()
        pltpu.sync_copy(acc.at[pl.ds(s * WCHD, WCHD)],
                        out_hbm.at[c, cc, pl.ds(s * WCHD, WCHD)])
        plsc.subcore_barrier()


def _make_sc_agg(ncc):
    f32 = jnp.float32
    i32 = jnp.int32
    fn = pl.kernel(
        functools.partial(_agg_body, ncc),
        out_type=jax.ShapeDtypeStruct((2, ncc, NROW, 32), f32),
        mesh=_MESH,
        compiler_params=_SC_PARAMS,
        scratch_types=[
            pltpu.VMEM((SB,), i32),
            pltpu.VMEM((SB,), i32),
        ] + [pltpu.VMEM((K,), i32) for _ in range(2 * GK)]
          + [pltpu.VMEM((K, 32), f32) for _ in range(GK)]
          + [pltpu.SemaphoreType.DMA, pltpu.SemaphoreType.DMA,
             pltpu.VMEM_SHARED((ACC_ROWS, 32), f32)],
    )

    def run(srcp, dstp, hp2):
        zeros = jnp.zeros((ZCHD, 32), f32)
        return fn(srcp, dstp, hp2, zeros)

    return run


_SC_AGG = {ncc: _make_sc_agg(ncc) for ncc in (3, 5, 10)}


TAIL_BM = 64


def _tail_body(g_ref, tt_ref, wg1_ref, bg1_ref, wg2_ref, bg2_ref,
               wc2_ref, bc_ref, wxt3_ref, bxt_ref,
               wf1_ref, bf1_ref, wf2_ref, bf2_ref, wo_ref, bo_ref,
               out_ref):
    f32 = jnp.float32
    bm = TAIL_BM
    g = g_ref[...]
    g = jnp.where(jnp.isfinite(g), g, 0.0)
    g1 = jax.nn.relu(jnp.dot(g, wg1_ref[...], preferred_element_type=f32)
                     + bg1_ref[...])
    g2 = jnp.dot(g1, wg2_ref[...], preferred_element_type=f32) + bg2_ref[...]

    xt = bxt_ref[...] * jnp.ones((bm, 1), f32)
    for t in range(12):
        ct = bc_ref[...] * jnp.ones((bm, 1), f32)
        for k in range(8):
            ct = ct + jnp.dot(tt_ref[:, t + k, :], wc2_ref[k],
                              preferred_element_type=f32)
        xt = xt + jnp.dot(ct, wxt3_ref[t], preferred_element_type=f32)

    h = jnp.dot(g2, wf1_ref[0:128, :], preferred_element_type=f32)
    h = h + jnp.dot(xt, wf1_ref[128:256, :], preferred_element_type=f32)
    h = jax.nn.relu(h + bf1_ref[...])
    h = jax.nn.relu(jnp.dot(h, wf2_ref[...], preferred_element_type=f32)
                    + bf2_ref[...])
    out_ref[...] = jnp.dot(h, wo_ref[...], preferred_element_type=f32) + bo_ref[...]


def _tail(g, target, Wg1, bg1, Wg2, bg2, Wc, bc, Wxt, bxt,
          Wf1, bf1, Wf2, bf2, Wo, bo):
    tt = jnp.transpose(target, (0, 2, 1))
    wc2 = jnp.transpose(Wc, (2, 1, 0))
    wxt3 = jnp.transpose(Wxt.reshape(32, 12, 128), (1, 0, 2))
    wg1p = jnp.pad(Wg1, ((0, 320 - Wg1.shape[0]), (0, 0)))
    bm = TAIL_BM
    row_spec = lambda minor: pl.BlockSpec((bm,) + minor, lambda i: (i,) + (0,) * len(minor))
    full = lambda a: pl.BlockSpec(a.shape, lambda i: (0,) * a.ndim)
    weights = [wg1p, bg1.reshape(1, -1), Wg2, bg2.reshape(1, -1),
               wc2, bc.reshape(1, -1), wxt3, bxt.reshape(1, -1),
               Wf1, bf1.reshape(1, -1), Wf2, bf2.reshape(1, -1),
               Wo, bo.reshape(1, -1)]
    return pl.pallas_call(
        _tail_body,
        grid=(B // bm,),
        in_specs=[row_spec((g.shape[1],)), row_spec((19, 750))]
                 + [full(w) for w in weights],
        out_specs=row_spec((1,)),
        out_shape=jax.ShapeDtypeStruct((B, 1), jnp.float32),
    )(g, tt, *weights)


TCBLK = 896
TCGRID = NROW // TCBLK


def _rows(minor):
    return pl.BlockSpec((TCBLK,) + minor, lambda i: (i,) + (0,) * len(minor))


def _full(a):
    return pl.BlockSpec(a.shape, lambda i: (0,) * a.ndim)


def _chunks(ncc):
    return pl.BlockSpec((ncc, TCBLK, 32), lambda i: (0, i, 0))


def _chunks2(ncc):
    return pl.BlockSpec((2, ncc, TCBLK, 32), lambda i: (0, 0, i, 0))


def _cat(ag):
    return jnp.concatenate([ag[c] for c in range(ag.shape[0])], axis=1)


def _tc1_body(x_ref, deg_ref, w_ref, hp_ref, dinv_ref):
    deg = deg_ref[0, :, 0:1] + deg_ref[1, :, 0:1] + 1.0
    dinv = lax.rsqrt(deg)
    h = jnp.dot(x_ref[...], w_ref[...], preferred_element_type=jnp.float32)
    h = h * dinv
    for c in range(hp_ref.shape[0]):
        hp_ref[c] = h[:, c * 32:(c + 1) * 32]
    dinv_ref[...] = dinv


def _tc_mid_body(ncc_out, a_ref, hp_ref, dinv_ref, w_ref, b_ref, out_ref):
    ag = _cat(a_ref[0] + a_ref[1]) + _cat(hp_ref[...])
    dinv = dinv_ref[...]
    h = jax.nn.relu(dinv * ag + b_ref[...])
    h2 = jnp.dot(h, w_ref[...], preferred_element_type=jnp.float32) * dinv
    for c in range(ncc_out):
        out_ref[c] = h2[:, c * 32:(c + 1) * 32]


def _tc4_body(a_ref, hp_ref, dinv_ref, b_ref, out_ref):
    ag = _cat(a_ref[0] + a_ref[1]) + _cat(hp_ref[...])
    out_ref[...] = jax.nn.relu(dinv_ref[...] * ag + b_ref[...])


def _tcseg_body(h_ref, ids_ref, out_ref, acc_ref, prev_ref):
    step = pl.program_id(0)

    @pl.when(step == 0)
    def _():
        out_ref[...] = jnp.full(out_ref.shape, -jnp.inf, jnp.float32)
        acc_ref[...] = jnp.full(acc_ref.shape, -jnp.inf, jnp.float32)
        prev_ref[0] = ids_ref[0, 0]

    def rowbody(r, carry):
        acc, prev = carry
        rid = ids_ref[step, r]
        ch = rid != prev

        @pl.when(ch)
        def _():
            out_ref[pl.ds(prev, 1), :] = acc

        row = h_ref[pl.ds(r, 1), :]
        acc = jnp.where(ch, row, jnp.maximum(acc, row))
        return acc, rid

    acc, prev = lax.fori_loop(0, TCBLK, rowbody,
                              (acc_ref[...], prev_ref[0]))
    acc_ref[...] = acc
    prev_ref[0] = prev

    @pl.when(step == TCGRID - 1)
    def _():
        out_ref[pl.ds(prev, 1), :] = acc


def kernel(x, edge_index, batch, target, W1, b1, W2, b2, W3, b3, Wg1, bg1,
           Wg2, bg2, Wc, bc, Wxt, bxt, Wf1, bf1, Wf2, bf2, Wo, bo):
    f32 = jnp.float32
    n = x.shape[0]
    srcp = jnp.concatenate(
        [edge_index[0], jnp.zeros((EPAD - E,), jnp.int32)]).reshape(32, EPT)
    dstp = jnp.concatenate(
        [edge_index[1], jnp.full((EPAD - E,), PADV, jnp.int32)]).reshape(32, EPT)
    xp = jnp.pad(x, ((0, NROW - n), (0, 0)))
    batchp = jnp.concatenate([batch, jnp.full((NROW - n,), B, jnp.int32)])

    w1p = jnp.pad(W1, ((0, 0), (0, 96 - 78)))
    w2p = jnp.pad(W2, ((0, 96 - 78), (0, 160 - 156)))
    w3p = jnp.pad(W3, ((0, 160 - 156), (0, 320 - 312)))
    b1p = jnp.pad(b1, (0, 96 - 78)).reshape(1, -1)
    b2p = jnp.pad(b2, (0, 160 - 156)).reshape(1, -1)
    b3p = jnp.pad(b3, (0, 320 - 312)).reshape(1, -1)

    degp = _sc_degree(dstp)

    hp1, dinv = pl.pallas_call(
        _tc1_body,
        grid=(TCGRID,),
        in_specs=[_rows((78,)), pl.BlockSpec((2, TCBLK, 16), lambda i: (0, i, 0)),
                  _full(w1p)],
        out_specs=[_chunks(3), _rows((1,))],
        out_shape=[jax.ShapeDtypeStruct((3, NROW, 32), f32),
                   jax.ShapeDtypeStruct((NROW, 1), f32)],
    )(xp, degp, w1p)

    agg1 = _SC_AGG[3](srcp, dstp, hp1.reshape(3 * NROW, 32))

    hp2 = pl.pallas_call(
        functools.partial(_tc_mid_body, 5),
        grid=(TCGRID,),
        in_specs=[_chunks2(3), _chunks(3), _rows((1,)), _full(w2p), _full(b1p)],
        out_specs=_chunks(5),
        out_shape=jax.ShapeDtypeStruct((5, NROW, 32), f32),
    )(agg1, hp1, dinv, w2p, b1p)

    agg2 = _SC_AGG[5](srcp, dstp, hp2.reshape(5 * NROW, 32))

    hp3 = pl.pallas_call(
        functools.partial(_tc_mid_body, 10),
        grid=(TCGRID,),
        in_specs=[_chunks2(5), _chunks(5), _rows((1,)), _full(w3p), _full(b2p)],
        out_specs=_chunks(10),
        out_shape=jax.ShapeDtypeStruct((10, NROW, 32), f32),
    )(agg2, hp2, dinv, w3p, b2p)

    agg3 = _SC_AGG[10](srcp, dstp, hp3.reshape(10 * NROW, 32))

    h3f = pl.pallas_call(
        _tc4_body,
        grid=(TCGRID,),
        in_specs=[_chunks2(10), _chunks(10), _rows((1,)), _full(b3p)],
        out_specs=_rows((320,)),
        out_shape=jax.ShapeDtypeStruct((NROW, 320), f32),
    )(agg3, hp3, dinv, b3p)

    g = pl.pallas_call(
        _tcseg_body,
        grid=(TCGRID,),
        in_specs=[_rows((320,)),
                  pl.BlockSpec((TCGRID, TCBLK), lambda i: (0, 0),
                               memory_space=pltpu.SMEM)],
        out_specs=pl.BlockSpec((B + 8, 320), lambda i: (0, 0)),
        out_shape=jax.ShapeDtypeStruct((B + 8, 320), f32),
        scratch_shapes=[pltpu.VMEM((1, 320), f32), pltpu.SMEM((1,), jnp.int32)],
    )(h3f, batchp.reshape(TCGRID, TCBLK))

    return _tail(g, target, Wg1, bg1, Wg2, bg2, Wc, bc, Wxt, bxt,
                 Wf1, bf1, Wf2, bf2, Wo, bo)


# speedup vs baseline: 7.8909x; 1.1311x over previous
import functools

import jax
import jax.numpy as jnp
from jax import lax
from jax.experimental import pallas as pl
from jax.experimental.pallas import tpu as pltpu
from jax.experimental.pallas import tpu_sc as plsc

N = 50000
B = 512
E = 800000
EPT = 25088
EPAD = 32 * EPT
PADV = 1 << 30
K = 128
NROW = 50176


DUMP = NROW
ACC_ROWS = NROW + 32
ZCHD = ACC_ROWS // 16
WCHD = NROW // 16
NBATCH = EPT // K

_MESH = plsc.VectorSubcoreMesh(core_axis_name="c", subcore_axis_name="s")
_SC_PARAMS = pltpu.CompilerParams(use_tc_tiling_on_sc=False)


SB = 3584
NBLK = EPT // SB
BPB = SB // K


def _deg_body(dst_hbm, zeros_hbm, ones_hbm, out_hbm, dbuf, fdst, ones, acc):
    c = lax.axis_index("c")
    s = lax.axis_index("s")
    wid = s * 2 + c
    pltpu.sync_copy(ones_hbm, ones)
    pltpu.sync_copy(zeros_hbm, acc.at[pl.ds(s * ZCHD, ZCHD)])
    plsc.subcore_barrier()

    def block(blk, carry):
        pltpu.sync_copy(dst_hbm.at[wid, pl.ds(blk * SB, SB)], dbuf)

        def batch(j, carry2):
            for t in range(8):
                dv = dbuf[pl.ds(j * K + t * 16, 16)]
                fdst[pl.ds(t * 16, 16)] = jnp.minimum(dv, DUMP)
            pltpu.sync_copy(ones, acc.at[fdst], add=True)
            return carry2

        lax.fori_loop(0, BPB, batch, 0)
        return carry

    lax.fori_loop(0, NBLK, block, 0)
    plsc.subcore_barrier()
    pltpu.sync_copy(acc.at[pl.ds(s * WCHD, WCHD)],
                    out_hbm.at[c, pl.ds(s * WCHD, WCHD)])


def _sc_degree(dstp):
    f32 = jnp.float32
    fn = pl.kernel(
        _deg_body,
        out_type=jax.ShapeDtypeStruct((2, NROW, 16), f32),
        mesh=_MESH,
        compiler_params=_SC_PARAMS,
        scratch_types=[
            pltpu.VMEM((SB,), jnp.int32),
            pltpu.VMEM((K,), jnp.int32),
            pltpu.VMEM((K, 16), f32),
            pltpu.VMEM_SHARED((ACC_ROWS, 16), f32),
        ],
    )
    zeros = jnp.zeros((ZCHD, 16), f32)
    ones = jnp.ones((K, 16), f32)
    return fn(dstp, zeros, ones)


GK = 4


def _agg_body(ncc, src_hbm, dst_hbm, hp_hbm, zeros_hbm, out_hbm,
              sbuf, dbuf, fs0, fs1, fs2, fs3, fd0, fd1, fd2, fd3,
              rb0, rb1, rb2, rb3, gsem, ssem, acc):
    FS = (fs0, fs1, fs2, fs3)
    FD = (fd0, fd1, fd2, fd3)
    RB = (rb0, rb1, rb2, rb3)
    c = lax.axis_index("c")
    s = lax.axis_index("s")
    wid = s * 2 + c
    for cc in range(ncc):
        pltpu.sync_copy(zeros_hbm, acc.at[pl.ds(s * ZCHD, ZCHD)])
        plsc.subcore_barrier()

        def block(blk, carry):
            pltpu.sync_copy(src_hbm.at[wid, pl.ds(blk * SB, SB)], sbuf)
            pltpu.sync_copy(dst_hbm.at[wid, pl.ds(blk * SB, SB)], dbuf)

            def group(gj, carry2):
                base = gj * (GK * K)
                for t in range(GK):
                    for u in range(8):
                        sv = sbuf[pl.ds(base + t * K + u * 16, 16)]
                        dv = dbuf[pl.ds(base + t * K + u * 16, 16)]
                        FS[t][pl.ds(u * 16, 16)] = sv + cc * NROW
                        FD[t][pl.ds(u * 16, 16)] = jnp.minimum(dv, DUMP)
                gds = [pltpu.async_copy(hp_hbm.at[FS[t]], RB[t], gsem)
                       for t in range(GK)]
                sds = []
                for t in range(GK):
                    gds[t].wait()
                    sds.append(pltpu.async_copy(RB[t], acc.at[FD[t]], ssem,
                                                add=True))
                for d in sds:
                    d.wait()
                return carry2

            lax.fori_loop(0, BPB // GK, group, 0)
            return carry

        lax.fori_loop(0, NBLK, block, 0)
        plsc.subcore_barrier()
        pltpu.sync_copy(acc.at[pl.ds(s * WCHD, WCHD)],
                        out_hbm.at[c, cc, pl.ds(s * WCHD, WCHD)])
        plsc.subcore_barrier()


def _make_sc_agg(ncc):
    f32 = jnp.float32
    i32 = jnp.int32
    fn = pl.kernel(
        functools.partial(_agg_body, ncc),
        out_type=jax.ShapeDtypeStruct((2, ncc, NROW, 32), f32),
        mesh=_MESH,
        compiler_params=_SC_PARAMS,
        scratch_types=[
            pltpu.VMEM((SB,), i32),
            pltpu.VMEM((SB,), i32),
        ] + [pltpu.VMEM((K,), i32) for _ in range(2 * GK)]
          + [pltpu.VMEM((K, 32), f32) for _ in range(GK)]
          + [pltpu.SemaphoreType.DMA, pltpu.SemaphoreType.DMA,
             pltpu.VMEM_SHARED((ACC_ROWS, 32), f32)],
    )

    def run(srcp, dstp, hp2):
        zeros = jnp.zeros((ZCHD, 32), f32)
        return fn(srcp, dstp, hp2, zeros)

    return run


_SC_AGG = {ncc: _make_sc_agg(ncc) for ncc in (3, 5, 10)}


TAIL_BM = 64


def _tail_body(g_ref, tt_ref, wg1_ref, bg1_ref, wg2_ref, bg2_ref,
               wc2_ref, bc_ref, wxt3_ref, bxt_ref,
               wf1_ref, bf1_ref, wf2_ref, bf2_ref, wo_ref, bo_ref,
               out_ref):
    f32 = jnp.float32
    bm = TAIL_BM
    g = g_ref[...]
    g = jnp.where(jnp.isfinite(g), g, 0.0)
    g1 = jax.nn.relu(jnp.dot(g, wg1_ref[...], preferred_element_type=f32)
                     + bg1_ref[...])
    g2 = jnp.dot(g1, wg2_ref[...], preferred_element_type=f32) + bg2_ref[...]

    xt = bxt_ref[...] * jnp.ones((bm, 1), f32)
    for t in range(12):
        ct = bc_ref[...] * jnp.ones((bm, 1), f32)
        for k in range(8):
            ct = ct + jnp.dot(tt_ref[:, t + k, :], wc2_ref[k],
                              preferred_element_type=f32)
        xt = xt + jnp.dot(ct, wxt3_ref[t], preferred_element_type=f32)

    h = jnp.dot(g2, wf1_ref[0:128, :], preferred_element_type=f32)
    h = h + jnp.dot(xt, wf1_ref[128:256, :], preferred_element_type=f32)
    h = jax.nn.relu(h + bf1_ref[...])
    h = jax.nn.relu(jnp.dot(h, wf2_ref[...], preferred_element_type=f32)
                    + bf2_ref[...])
    out_ref[...] = jnp.dot(h, wo_ref[...], preferred_element_type=f32) + bo_ref[...]


def _tail(g, target, Wg1, bg1, Wg2, bg2, Wc, bc, Wxt, bxt,
          Wf1, bf1, Wf2, bf2, Wo, bo):
    tt = jnp.transpose(target, (0, 2, 1))
    wc2 = jnp.transpose(Wc, (2, 1, 0))
    wxt3 = jnp.transpose(Wxt.reshape(32, 12, 128), (1, 0, 2))
    wg1p = jnp.pad(Wg1, ((0, 320 - Wg1.shape[0]), (0, 0)))
    bm = TAIL_BM
    row_spec = lambda minor: pl.BlockSpec((bm,) + minor, lambda i: (i,) + (0,) * len(minor))
    full = lambda a: pl.BlockSpec(a.shape, lambda i: (0,) * a.ndim)
    weights = [wg1p, bg1.reshape(1, -1), Wg2, bg2.reshape(1, -1),
               wc2, bc.reshape(1, -1), wxt3, bxt.reshape(1, -1),
               Wf1, bf1.reshape(1, -1), Wf2, bf2.reshape(1, -1),
               Wo, bo.reshape(1, -1)]
    return pl.pallas_call(
        _tail_body,
        grid=(B // bm,),
        in_specs=[row_spec((g.shape[1],)), row_spec((19, 750))]
                 + [full(w) for w in weights],
        out_specs=row_spec((1,)),
        out_shape=jax.ShapeDtypeStruct((B, 1), jnp.float32),
    )(g, tt, *weights)


TCBLK = 896
TCGRID = NROW // TCBLK


def _rows(minor):
    return pl.BlockSpec((TCBLK,) + minor, lambda i: (i,) + (0,) * len(minor))


def _full(a):
    return pl.BlockSpec(a.shape, lambda i: (0,) * a.ndim)


def _chunks(ncc):
    return pl.BlockSpec((ncc, TCBLK, 32), lambda i: (0, i, 0))


def _chunks2(ncc):
    return pl.BlockSpec((2, ncc, TCBLK, 32), lambda i: (0, 0, i, 0))


def _cat(ag):
    return jnp.concatenate([ag[c] for c in range(ag.shape[0])], axis=1)


def _tc1_body(x_ref, deg_ref, w_ref, hp_ref, dinv_ref):
    deg = deg_ref[0, :, 0:1] + deg_ref[1, :, 0:1] + 1.0
    dinv = lax.rsqrt(deg)
    h = jnp.dot(x_ref[...], w_ref[...], preferred_element_type=jnp.float32)
    h = h * dinv
    for c in range(hp_ref.shape[0]):
        hp_ref[c] = h[:, c * 32:(c + 1) * 32]
    dinv_ref[...] = dinv


def _tc_mid_body(ncc_out, a_ref, hp_ref, dinv_ref, w_ref, b_ref, out_ref):
    ag = _cat(a_ref[0] + a_ref[1]) + _cat(hp_ref[...])
    dinv = dinv_ref[...]
    h = jax.nn.relu(dinv * ag + b_ref[...])
    h2 = jnp.dot(h, w_ref[...], preferred_element_type=jnp.float32) * dinv
    for c in range(ncc_out):
        out_ref[c] = h2[:, c * 32:(c + 1) * 32]


def _tc4_body(a_ref, hp_ref, dinv_ref, b_ref, out_ref):
    ag = _cat(a_ref[0] + a_ref[1]) + _cat(hp_ref[...])
    out_ref[...] = jax.nn.relu(dinv_ref[...] * ag + b_ref[...])


def _tcseg_body(h_ref, ids_ref, out_ref, acc_ref, prev_ref):
    step = pl.program_id(0)

    i32 = jnp.int32
    minf = jnp.full(acc_ref.shape, -jnp.inf, jnp.float32)

    @pl.when(step == 0)
    def _():
        out_ref[...] = jnp.full(out_ref.shape, -jnp.inf, jnp.float32)
        acc_ref[...] = minf
        prev_ref[0] = ids_ref[0, 0]

    iota32 = lax.broadcasted_iota(i32, (32, 1), 0)
    for g in range(TCBLK // 32):
        blk = h_ref[g * 32:(g + 1) * 32, :]
        a_id = ids_ref[step, g * 32]
        b_id = ids_ref[step, g * 32 + 31]

        @pl.when(a_id == b_id)
        def _(blk=blk, a_id=a_id):
            @pl.when(a_id != prev_ref[0])
            def _():
                out_ref[pl.ds(prev_ref[0], 1), :] = acc_ref[...]
                acc_ref[...] = minf

            acc_ref[...] = jnp.maximum(
                acc_ref[...], jnp.max(blk, axis=0, keepdims=True))
            prev_ref[0] = a_id

        @pl.when(a_id != b_id)
        def _(blk=blk, g=g):
            def run_body(a):
                ida = ids_ref[step, g * 32 + a]

                def scan_cond(bq):
                    return (bq < 32) & (ids_ref[step, g * 32 + jnp.minimum(bq, 31)] == ida)

                bq = lax.while_loop(scan_cond, lambda v: v + 1, a + 1)

                @pl.when(ida != prev_ref[0])
                def _():
                    out_ref[pl.ds(prev_ref[0], 1), :] = acc_ref[...]
                    acc_ref[...] = minf

                mask = (iota32 >= a) & (iota32 < bq)
                mmax = jnp.max(jnp.where(mask, blk, -jnp.inf), axis=0,
                               keepdims=True)
                acc_ref[...] = jnp.maximum(acc_ref[...], mmax)
                prev_ref[0] = ida
                return bq

            lax.while_loop(lambda a: a < 32, run_body, jnp.int32(0))

    @pl.when(step == TCGRID - 1)
    def _():
        out_ref[pl.ds(prev_ref[0], 1), :] = acc_ref[...]


def kernel(x, edge_index, batch, target, W1, b1, W2, b2, W3, b3, Wg1, bg1,
           Wg2, bg2, Wc, bc, Wxt, bxt, Wf1, bf1, Wf2, bf2, Wo, bo):
    f32 = jnp.float32
    n = x.shape[0]
    srcp = jnp.concatenate(
        [edge_index[0], jnp.zeros((EPAD - E,), jnp.int32)]).reshape(32, EPT)
    dstp = jnp.concatenate(
        [edge_index[1], jnp.full((EPAD - E,), PADV, jnp.int32)]).reshape(32, EPT)
    xp = jnp.pad(x, ((0, NROW - n), (0, 0)))
    batchp = jnp.concatenate([batch, jnp.full((NROW - n,), B, jnp.int32)])

    w1p = jnp.pad(W1, ((0, 0), (0, 96 - 78)))
    w2p = jnp.pad(W2, ((0, 96 - 78), (0, 160 - 156)))
    w3p = jnp.pad(W3, ((0, 160 - 156), (0, 320 - 312)))
    b1p = jnp.pad(b1, (0, 96 - 78)).reshape(1, -1)
    b2p = jnp.pad(b2, (0, 160 - 156)).reshape(1, -1)
    b3p = jnp.pad(b3, (0, 320 - 312)).reshape(1, -1)

    degp = _sc_degree(dstp)

    hp1, dinv = pl.pallas_call(
        _tc1_body,
        grid=(TCGRID,),
        in_specs=[_rows((78,)), pl.BlockSpec((2, TCBLK, 16), lambda i: (0, i, 0)),
                  _full(w1p)],
        out_specs=[_chunks(3), _rows((1,))],
        out_shape=[jax.ShapeDtypeStruct((3, NROW, 32), f32),
                   jax.ShapeDtypeStruct((NROW, 1), f32)],
    )(xp, degp, w1p)

    agg1 = _SC_AGG[3](srcp, dstp, hp1.reshape(3 * NROW, 32))

    hp2 = pl.pallas_call(
        functools.partial(_tc_mid_body, 5),
        grid=(TCGRID,),
        in_specs=[_chunks2(3), _chunks(3), _rows((1,)), _full(w2p), _full(b1p)],
        out_specs=_chunks(5),
        out_shape=jax.ShapeDtypeStruct((5, NROW, 32), f32),
    )(agg1, hp1, dinv, w2p, b1p)

    agg2 = _SC_AGG[5](srcp, dstp, hp2.reshape(5 * NROW, 32))

    hp3 = pl.pallas_call(
        functools.partial(_tc_mid_body, 10),
        grid=(TCGRID,),
        in_specs=[_chunks2(5), _chunks(5), _rows((1,)), _full(w3p), _full(b2p)],
        out_specs=_chunks(10),
        out_shape=jax.ShapeDtypeStruct((10, NROW, 32), f32),
    )(agg2, hp2, dinv, w3p, b2p)

    agg3 = _SC_AGG[10](srcp, dstp, hp3.reshape(10 * NROW, 32))

    h3f = pl.pallas_call(
        _tc4_body,
        grid=(TCGRID,),
        in_specs=[_chunks2(10), _chunks(10), _rows((1,)), _full(b3p)],
        out_specs=_rows((320,)),
        out_shape=jax.ShapeDtypeStruct((NROW, 320), f32),
    )(agg3, hp3, dinv, b3p)

    g = pl.pallas_call(
        _tcseg_body,
        grid=(TCGRID,),
        in_specs=[_rows((320,)),
                  pl.BlockSpec((TCGRID, TCBLK), lambda i: (0, 0),
                               memory_space=pltpu.SMEM)],
        out_specs=pl.BlockSpec((B + 8, 320), lambda i: (0, 0)),
        out_shape=jax.ShapeDtypeStruct((B + 8, 320), f32),
        scratch_shapes=[pltpu.VMEM((1, 320), f32), pltpu.SMEM((1,), jnp.int32)],
    )(h3f, batchp.reshape(TCGRID, TCBLK))

    return _tail(g, target, Wg1, bg1, Wg2, bg2, Wc, bc, Wxt, bxt,
                 Wf1, bf1, Wf2, bf2, Wo, bo)


# speedup vs baseline: 8.0958x; 1.0260x over previous
import functools

import jax
import jax.numpy as jnp
from jax import lax
from jax.experimental import pallas as pl
from jax.experimental.pallas import tpu as pltpu
from jax.experimental.pallas import tpu_sc as plsc

N = 50000
B = 512
E = 800000
EPT = 25088
EPAD = 32 * EPT
PADV = 1 << 30
K = 128
NROW = 50176


DUMP = NROW
ACC_ROWS = NROW + 32
ZCHD = ACC_ROWS // 16
WCHD = NROW // 16
NBATCH = EPT // K

_MESH = plsc.VectorSubcoreMesh(core_axis_name="c", subcore_axis_name="s")
_SC_PARAMS = pltpu.CompilerParams(use_tc_tiling_on_sc=False)


SB = 3584
NBLK = EPT // SB
BPB = SB // K


def _deg_body(dst_hbm, zeros_hbm, ones_hbm, out_hbm, dbuf, fdst, ones, acc):
    c = lax.axis_index("c")
    s = lax.axis_index("s")
    wid = s * 2 + c
    pltpu.sync_copy(ones_hbm, ones)
    pltpu.sync_copy(zeros_hbm, acc.at[pl.ds(s * ZCHD, ZCHD)])
    plsc.subcore_barrier()

    def block(blk, carry):
        pltpu.sync_copy(dst_hbm.at[wid, pl.ds(blk * SB, SB)], dbuf)

        def batch(j, carry2):
            for t in range(8):
                dv = dbuf[pl.ds(j * K + t * 16, 16)]
                fdst[pl.ds(t * 16, 16)] = jnp.minimum(dv, DUMP)
            pltpu.sync_copy(ones, acc.at[fdst], add=True)
            return carry2

        lax.fori_loop(0, BPB, batch, 0)
        return carry

    lax.fori_loop(0, NBLK, block, 0)
    plsc.subcore_barrier()
    pltpu.sync_copy(acc.at[pl.ds(s * WCHD, WCHD)],
                    out_hbm.at[c, pl.ds(s * WCHD, WCHD)])


def _sc_degree(dstp):
    f32 = jnp.float32
    fn = pl.kernel(
        _deg_body,
        out_type=jax.ShapeDtypeStruct((2, NROW, 16), f32),
        mesh=_MESH,
        compiler_params=_SC_PARAMS,
        scratch_types=[
            pltpu.VMEM((SB,), jnp.int32),
            pltpu.VMEM((K,), jnp.int32),
            pltpu.VMEM((K, 16), f32),
            pltpu.VMEM_SHARED((ACC_ROWS, 16), f32),
        ],
    )
    zeros = jnp.zeros((ZCHD, 16), f32)
    ones = jnp.ones((K, 16), f32)
    return fn(dstp, zeros, ones)


GK = 4


def _agg_body(ncc, src_hbm, dst_hbm, hp_hbm, zeros_hbm, out_hbm,
              sbuf, dbuf, fs0, fs1, fs2, fs3, fd0, fd1, fd2, fd3,
              rb0, rb1, rb2, rb3, gsem, ssem, acc):
    FS = (fs0, fs1, fs2, fs3)
    FD = (fd0, fd1, fd2, fd3)
    RB = (rb0, rb1, rb2, rb3)
    c = lax.axis_index("c")
    s = lax.axis_index("s")
    wid = s * 2 + c
    for cc in range(ncc):
        pltpu.sync_copy(zeros_hbm, acc.at[pl.ds(s * ZCHD, ZCHD)])
        plsc.subcore_barrier()

        def block(blk, carry):
            pltpu.sync_copy(src_hbm.at[wid, pl.ds(blk * SB, SB)], sbuf)
            pltpu.sync_copy(dst_hbm.at[wid, pl.ds(blk * SB, SB)], dbuf)

            def group(gj, carry2):
                base = gj * (GK * K)
                for t in range(GK):
                    for u in range(8):
                        sv = sbuf[pl.ds(base + t * K + u * 16, 16)]
                        dv = dbuf[pl.ds(base + t * K + u * 16, 16)]
                        FS[t][pl.ds(u * 16, 16)] = sv + cc * NROW
                        FD[t][pl.ds(u * 16, 16)] = jnp.minimum(dv, DUMP)
                gds = [pltpu.async_copy(hp_hbm.at[FS[t]], RB[t], gsem)
                       for t in range(GK)]
                sds = []
                for t in range(GK):
                    gds[t].wait()
                    sds.append(pltpu.async_copy(RB[t], acc.at[FD[t]], ssem,
                                                add=True))
                for d in sds:
                    d.wait()
                return carry2

            lax.fori_loop(0, BPB // GK, group, 0)
            return carry

        lax.fori_loop(0, NBLK, block, 0)
        plsc.subcore_barrier()
        pltpu.sync_copy(acc.at[pl.ds(s * WCHD, WCHD)],
                        out_hbm.at[c, cc, pl.ds(s * WCHD, WCHD)])
        plsc.subcore_barrier()


def _make_sc_agg(ncc):
    f32 = jnp.float32
    i32 = jnp.int32
    fn = pl.kernel(
        functools.partial(_agg_body, ncc),
        out_type=jax.ShapeDtypeStruct((2, ncc, NROW, 32), f32),
        mesh=_MESH,
        compiler_params=_SC_PARAMS,
        scratch_types=[
            pltpu.VMEM((SB,), i32),
            pltpu.VMEM((SB,), i32),
        ] + [pltpu.VMEM((K,), i32) for _ in range(2 * GK)]
          + [pltpu.VMEM((K, 32), f32) for _ in range(GK)]
          + [pltpu.SemaphoreType.DMA, pltpu.SemaphoreType.DMA,
             pltpu.VMEM_SHARED((ACC_ROWS, 32), f32)],
    )

    def run(srcp, dstp, hp2):
        zeros = jnp.zeros((ZCHD, 32), f32)
        return fn(srcp, dstp, hp2, zeros)

    return run


_SC_AGG = {ncc: _make_sc_agg(ncc) for ncc in (3, 5, 10)}


TAIL_BM = 64


def _tail_body(g_ref, tt_ref, wg1_ref, bg1_ref, wg2_ref, bg2_ref,
               wc2_ref, bc_ref, wxt3_ref, bxt_ref,
               wf1_ref, bf1_ref, wf2_ref, bf2_ref, wo_ref, bo_ref,
               out_ref):
    f32 = jnp.float32
    bm = TAIL_BM
    g = g_ref[...]
    g = jnp.where(jnp.isfinite(g), g, 0.0)
    g1 = jax.nn.relu(jnp.dot(g, wg1_ref[...], preferred_element_type=f32)
                     + bg1_ref[...])
    g2 = jnp.dot(g1, wg2_ref[...], preferred_element_type=f32) + bg2_ref[...]

    xt = bxt_ref[...] * jnp.ones((bm, 1), f32)
    for t in range(12):
        ct = bc_ref[...] * jnp.ones((bm, 1), f32)
        for k in range(8):
            ct = ct + jnp.dot(tt_ref[:, t + k, :], wc2_ref[k],
                              preferred_element_type=f32)
        xt = xt + jnp.dot(ct, wxt3_ref[t], preferred_element_type=f32)

    h = jnp.dot(g2, wf1_ref[0:128, :], preferred_element_type=f32)
    h = h + jnp.dot(xt, wf1_ref[128:256, :], preferred_element_type=f32)
    h = jax.nn.relu(h + bf1_ref[...])
    h = jax.nn.relu(jnp.dot(h, wf2_ref[...], preferred_element_type=f32)
                    + bf2_ref[...])
    out_ref[...] = jnp.dot(h, wo_ref[...], preferred_element_type=f32) + bo_ref[...]


def _tail(g, target, Wg1, bg1, Wg2, bg2, Wc, bc, Wxt, bxt,
          Wf1, bf1, Wf2, bf2, Wo, bo):
    tt = jnp.transpose(target, (0, 2, 1))
    wc2 = jnp.transpose(Wc, (2, 1, 0))
    wxt3 = jnp.transpose(Wxt.reshape(32, 12, 128), (1, 0, 2))
    wg1p = jnp.pad(Wg1, ((0, 320 - Wg1.shape[0]), (0, 0)))
    bm = TAIL_BM
    row_spec = lambda minor: pl.BlockSpec((bm,) + minor, lambda i: (i,) + (0,) * len(minor))
    full = lambda a: pl.BlockSpec(a.shape, lambda i: (0,) * a.ndim)
    weights = [wg1p, bg1.reshape(1, -1), Wg2, bg2.reshape(1, -1),
               wc2, bc.reshape(1, -1), wxt3, bxt.reshape(1, -1),
               Wf1, bf1.reshape(1, -1), Wf2, bf2.reshape(1, -1),
               Wo, bo.reshape(1, -1)]
    return pl.pallas_call(
        _tail_body,
        grid=(B // bm,),
        in_specs=[row_spec((g.shape[1],)), row_spec((19, 750))]
                 + [full(w) for w in weights],
        out_specs=row_spec((1,)),
        out_shape=jax.ShapeDtypeStruct((B, 1), jnp.float32),
    )(g, tt, *weights)


TCBLK = 896
TCGRID = NROW // TCBLK


def _rows(minor):
    return pl.BlockSpec((TCBLK,) + minor, lambda i: (i,) + (0,) * len(minor))


def _full(a):
    return pl.BlockSpec(a.shape, lambda i: (0,) * a.ndim)


def _chunks(ncc):
    return pl.BlockSpec((ncc, TCBLK, 32), lambda i: (0, i, 0))


def _chunks2(ncc):
    return pl.BlockSpec((2, ncc, TCBLK, 32), lambda i: (0, 0, i, 0))


def _cat(ag):
    return jnp.concatenate([ag[c] for c in range(ag.shape[0])], axis=1)


def _tc1_body(x_ref, deg_ref, w_ref, hp_ref, dinv_ref):
    deg = deg_ref[0, :, 0:1] + deg_ref[1, :, 0:1] + 1.0
    dinv = lax.rsqrt(deg)
    h = jnp.dot(x_ref[...], w_ref[...], preferred_element_type=jnp.float32)
    h = h * dinv
    for c in range(hp_ref.shape[0]):
        hp_ref[c] = h[:, c * 32:(c + 1) * 32]
    dinv_ref[...] = dinv


def _tc_mid_body(ncc_out, a_ref, hp_ref, dinv_ref, w_ref, b_ref, out_ref):
    ag = _cat(a_ref[0] + a_ref[1]) + _cat(hp_ref[...])
    dinv = dinv_ref[...]
    h = jax.nn.relu(dinv * ag + b_ref[...])
    h2 = jnp.dot(h, w_ref[...], preferred_element_type=jnp.float32) * dinv
    for c in range(ncc_out):
        out_ref[c] = h2[:, c * 32:(c + 1) * 32]


def _tcseg_body(a_ref, hp_ref, dinv_ref, b_ref, ids_ref, out_ref,
                acc_ref, prev_ref):
    step = pl.program_id(0)
    ag = _cat(a_ref[0] + a_ref[1]) + _cat(hp_ref[...])
    h = jax.nn.relu(dinv_ref[...] * ag + b_ref[...])

    i32 = jnp.int32
    minf = jnp.full(acc_ref.shape, -jnp.inf, jnp.float32)

    @pl.when(step == 0)
    def _():
        out_ref[...] = jnp.full(out_ref.shape, -jnp.inf, jnp.float32)
        acc_ref[...] = minf
        prev_ref[0] = ids_ref[0, 0]

    iota32 = lax.broadcasted_iota(i32, (32, 1), 0)
    for g in range(TCBLK // 32):
        blk = h[g * 32:(g + 1) * 32, :]
        a_id = ids_ref[step, g * 32]
        b_id = ids_ref[step, g * 32 + 31]

        @pl.when(a_id == b_id)
        def _(blk=blk, a_id=a_id):
            @pl.when(a_id != prev_ref[0])
            def _():
                out_ref[pl.ds(prev_ref[0], 1), :] = acc_ref[...]
                acc_ref[...] = minf

            acc_ref[...] = jnp.maximum(
                acc_ref[...], jnp.max(blk, axis=0, keepdims=True))
            prev_ref[0] = a_id

        @pl.when(a_id != b_id)
        def _(blk=blk, g=g):
            def run_body(a):
                ida = ids_ref[step, g * 32 + a]

                def scan_cond(bq):
                    return (bq < 32) & (ids_ref[step, g * 32 + jnp.minimum(bq, 31)] == ida)

                bq = lax.while_loop(scan_cond, lambda v: v + 1, a + 1)

                @pl.when(ida != prev_ref[0])
                def _():
                    out_ref[pl.ds(prev_ref[0], 1), :] = acc_ref[...]
                    acc_ref[...] = minf

                mask = (iota32 >= a) & (iota32 < bq)
                mmax = jnp.max(jnp.where(mask, blk, -jnp.inf), axis=0,
                               keepdims=True)
                acc_ref[...] = jnp.maximum(acc_ref[...], mmax)
                prev_ref[0] = ida
                return bq

            lax.while_loop(lambda a: a < 32, run_body, jnp.int32(0))

    @pl.when(step == TCGRID - 1)
    def _():
        out_ref[pl.ds(prev_ref[0], 1), :] = acc_ref[...]


def kernel(x, edge_index, batch, target, W1, b1, W2, b2, W3, b3, Wg1, bg1,
           Wg2, bg2, Wc, bc, Wxt, bxt, Wf1, bf1, Wf2, bf2, Wo, bo):
    f32 = jnp.float32
    n = x.shape[0]
    srcp = jnp.concatenate(
        [edge_index[0], jnp.zeros((EPAD - E,), jnp.int32)]).reshape(32, EPT)
    dstp = jnp.concatenate(
        [edge_index[1], jnp.full((EPAD - E,), PADV, jnp.int32)]).reshape(32, EPT)
    xp = jnp.pad(x, ((0, NROW - n), (0, 0)))
    batchp = jnp.concatenate([batch, jnp.full((NROW - n,), B, jnp.int32)])

    w1p = jnp.pad(W1, ((0, 0), (0, 96 - 78)))
    w2p = jnp.pad(W2, ((0, 96 - 78), (0, 160 - 156)))
    w3p = jnp.pad(W3, ((0, 160 - 156), (0, 320 - 312)))
    b1p = jnp.pad(b1, (0, 96 - 78)).reshape(1, -1)
    b2p = jnp.pad(b2, (0, 160 - 156)).reshape(1, -1)
    b3p = jnp.pad(b3, (0, 320 - 312)).reshape(1, -1)

    degp = _sc_degree(dstp)

    hp1, dinv = pl.pallas_call(
        _tc1_body,
        grid=(TCGRID,),
        in_specs=[_rows((78,)), pl.BlockSpec((2, TCBLK, 16), lambda i: (0, i, 0)),
                  _full(w1p)],
        out_specs=[_chunks(3), _rows((1,))],
        out_shape=[jax.ShapeDtypeStruct((3, NROW, 32), f32),
                   jax.ShapeDtypeStruct((NROW, 1), f32)],
    )(xp, degp, w1p)

    agg1 = _SC_AGG[3](srcp, dstp, hp1.reshape(3 * NROW, 32))

    hp2 = pl.pallas_call(
        functools.partial(_tc_mid_body, 5),
        grid=(TCGRID,),
        in_specs=[_chunks2(3), _chunks(3), _rows((1,)), _full(w2p), _full(b1p)],
        out_specs=_chunks(5),
        out_shape=jax.ShapeDtypeStruct((5, NROW, 32), f32),
    )(agg1, hp1, dinv, w2p, b1p)

    agg2 = _SC_AGG[5](srcp, dstp, hp2.reshape(5 * NROW, 32))

    hp3 = pl.pallas_call(
        functools.partial(_tc_mid_body, 10),
        grid=(TCGRID,),
        in_specs=[_chunks2(5), _chunks(5), _rows((1,)), _full(w3p), _full(b2p)],
        out_specs=_chunks(10),
        out_shape=jax.ShapeDtypeStruct((10, NROW, 32), f32),
    )(agg2, hp2, dinv, w3p, b2p)

    agg3 = _SC_AGG[10](srcp, dstp, hp3.reshape(10 * NROW, 32))

    g = pl.pallas_call(
        _tcseg_body,
        grid=(TCGRID,),
        in_specs=[_chunks2(10), _chunks(10), _rows((1,)), _full(b3p),
                  pl.BlockSpec((TCGRID, TCBLK), lambda i: (0, 0),
                               memory_space=pltpu.SMEM)],
        out_specs=pl.BlockSpec((B + 8, 320), lambda i: (0, 0)),
        out_shape=jax.ShapeDtypeStruct((B + 8, 320), f32),
        scratch_shapes=[pltpu.VMEM((1, 320), f32), pltpu.SMEM((1,), jnp.int32)],
    )(agg3, hp3, dinv, b3p, batchp.reshape(TCGRID, TCBLK))

    return _tail(g, target, Wg1, bg1, Wg2, bg2, Wc, bc, Wxt, bxt,
                 Wf1, bf1, Wf2, bf2, Wo, bo)
